# Initial kernel scaffold; baseline (speedup 1.0000x reference)
import jax
import jax.numpy as jnp
from jax.experimental import pallas as pl


def kernel(pred_rotmats, scores, grid_rotmats, medians, mads):
    raise NotImplementedError("write your pallas kernel here")


# speedup vs baseline: 2.9509x; 2.9509x over previous
import functools

import jax
import jax.numpy as jnp
from jax.experimental import pallas as pl

N_PSI = 24
N_CONES = 192
N_SO3 = N_CONES * N_PSI
BLOCK_B = 1024


def _fused_kernel(pred_ref, gridT_ref, scores_ref, stats_ref, out_ref):
    sim = jnp.dot(pred_ref[...], gridT_ref[...],
                  preferred_element_type=jnp.float32)
    rowmax = jnp.max(sim, axis=1, keepdims=True)
    jidx = jax.lax.broadcasted_iota(jnp.int32, sim.shape, 1)
    idx = jnp.min(jnp.where(sim == rowmax, jidx, N_SO3), axis=1, keepdims=True)
    cone = idx // N_PSI
    onehot = (cone == jax.lax.broadcasted_iota(
        jnp.int32, (1, N_CONES), 1)).astype(jnp.float32)
    st = jnp.dot(onehot, stats_ref[...],
                 preferred_element_type=jnp.float32)
    med = st[:, 0:1]
    mad = st[:, 1:2]
    out_ref[...] = (scores_ref[...] - med) / mad


@jax.jit
def kernel(pred_rotmats, scores, grid_rotmats, medians, mads):
    b = pred_rotmats.shape[0]
    pred_flat = pred_rotmats.reshape(b, 9)
    gridT = grid_rotmats.reshape(N_SO3, 9).T
    stats = jnp.stack([medians, mads], axis=1)
    scores2 = scores.reshape(b, 1)

    out = pl.pallas_call(
        _fused_kernel,
        grid=(b // BLOCK_B,),
        in_specs=[
            pl.BlockSpec((BLOCK_B, 9), lambda i: (i, 0)),
            pl.BlockSpec((9, N_SO3), lambda i: (0, 0)),
            pl.BlockSpec((BLOCK_B, 1), lambda i: (i, 0)),
            pl.BlockSpec((N_CONES, 2), lambda i: (0, 0)),
        ],
        out_specs=pl.BlockSpec((BLOCK_B, 1), lambda i: (i, 0)),
        out_shape=jax.ShapeDtypeStruct((b, 1), jnp.float32),
    )(pred_flat, gridT, scores2, stats)
    return out.reshape(b)


# speedup vs baseline: 3.2191x; 1.0909x over previous
import functools

import jax
import jax.numpy as jnp
from jax.experimental import pallas as pl

N_PSI = 24
N_CONES = 192
N_SO3 = N_CONES * N_PSI
BLOCK_B = 1024


def _fused_kernel(pred_ref, gridT_ref, scores_ref, stats_ref, out_ref):
    sim = jnp.dot(pred_ref[...], gridT_ref[...],
                  preferred_element_type=jnp.float32)
    conemax = sim[:, 0:N_CONES]
    for p in range(1, N_PSI):
        conemax = jnp.maximum(conemax, sim[:, p * N_CONES:(p + 1) * N_CONES])
    rowmax = jnp.max(conemax, axis=1, keepdims=True)
    cidx = jax.lax.broadcasted_iota(jnp.int32, (1, N_CONES), 1)
    cone = jnp.min(jnp.where(conemax == rowmax, cidx, N_CONES),
                   axis=1, keepdims=True)
    onehot = (cone == cidx).astype(jnp.float32)
    st = jnp.dot(onehot, stats_ref[...],
                 preferred_element_type=jnp.float32)
    med = st[:, 0:1]
    mad = st[:, 1:2]
    out_ref[...] = (scores_ref[...] - med) / mad


@jax.jit
def kernel(pred_rotmats, scores, grid_rotmats, medians, mads):
    b = pred_rotmats.shape[0]
    pred_flat = pred_rotmats.reshape(b, 9)
    gridT = grid_rotmats.reshape(N_CONES, N_PSI, 9).transpose(
        2, 1, 0).reshape(9, N_SO3)
    stats = jnp.stack([medians, mads], axis=1)
    scores2 = scores.reshape(b, 1)

    out = pl.pallas_call(
        _fused_kernel,
        grid=(b // BLOCK_B,),
        in_specs=[
            pl.BlockSpec((BLOCK_B, 9), lambda i: (i, 0)),
            pl.BlockSpec((9, N_SO3), lambda i: (0, 0)),
            pl.BlockSpec((BLOCK_B, 1), lambda i: (i, 0)),
            pl.BlockSpec((N_CONES, 2), lambda i: (0, 0)),
        ],
        out_specs=pl.BlockSpec((BLOCK_B, 1), lambda i: (i, 0)),
        out_shape=jax.ShapeDtypeStruct((b, 1), jnp.float32),
    )(pred_flat, gridT, scores2, stats)
    return out.reshape(b)


# speedup vs baseline: 4.2507x; 1.3204x over previous
import jax
import jax.numpy as jnp
from jax.experimental import pallas as pl

N_PSI = 24
N_CONES = 192
N_SO3 = N_CONES * N_PSI
BLOCK_B = 1024


def _fused_kernel(pred_ref, gridT_ref, scores_ref, med_ref, mad_ref, out_ref):
    sim = jnp.dot(pred_ref[...], gridT_ref[...],
                  preferred_element_type=jnp.float32)
    conemax = sim[:, 0:N_CONES]
    for p in range(1, N_PSI):
        conemax = jnp.maximum(conemax, sim[:, p * N_CONES:(p + 1) * N_CONES])
    rowmax = jnp.max(conemax, axis=1, keepdims=True)
    cidx = jax.lax.broadcasted_iota(jnp.int32, (1, N_CONES), 1)
    cone = jnp.min(jnp.where(conemax == rowmax, cidx, N_CONES),
                   axis=1, keepdims=True)
    onehot = (cone == cidx).astype(jnp.float32)
    stats = jnp.concatenate([med_ref[...], mad_ref[...]], axis=0)
    st = jax.lax.dot_general(
        stats, onehot, (((1,), (1,)), ((), ())),
        preferred_element_type=jnp.float32)
    out_ref[...] = (scores_ref[...] - st[0:1, :]) / st[1:2, :]


@jax.jit
def kernel(pred_rotmats, scores, grid_rotmats, medians, mads):
    b = pred_rotmats.shape[0]
    pred_flat = pred_rotmats.reshape(b, 9)
    gridT = grid_rotmats.reshape(N_CONES, N_PSI, 9).transpose(
        2, 1, 0).reshape(9, N_SO3)

    out = pl.pallas_call(
        _fused_kernel,
        grid=(b // BLOCK_B,),
        in_specs=[
            pl.BlockSpec((BLOCK_B, 9), lambda i: (i, 0)),
            pl.BlockSpec((9, N_SO3), lambda i: (0, 0)),
            pl.BlockSpec((1, BLOCK_B), lambda i: (0, i)),
            pl.BlockSpec((1, N_CONES), lambda i: (0, 0)),
            pl.BlockSpec((1, N_CONES), lambda i: (0, 0)),
        ],
        out_specs=pl.BlockSpec((1, BLOCK_B), lambda i: (0, i)),
        out_shape=jax.ShapeDtypeStruct((1, b), jnp.float32),
    )(pred_flat, gridT, scores.reshape(1, b),
      medians.reshape(1, N_CONES), mads.reshape(1, N_CONES))
    return out.reshape(b)


# speedup vs baseline: 4.2622x; 1.0027x over previous
import jax
import jax.numpy as jnp
from jax.experimental import pallas as pl
from jax.experimental.pallas import tpu as pltpu

N_PSI = 24
N_CONES = 192
N_SO3 = N_CONES * N_PSI
BLOCK_B = 1024


def _fused_kernel(pred_ref, gridT_ref, scores_ref, med_ref, mad_ref, out_ref):
    sim = jnp.dot(pred_ref[...], gridT_ref[...],
                  preferred_element_type=jnp.float32)
    conemax = sim[:, 0:N_CONES]
    for p in range(1, N_PSI):
        conemax = jnp.maximum(conemax, sim[:, p * N_CONES:(p + 1) * N_CONES])
    rowmax = jnp.max(conemax, axis=1, keepdims=True)
    cidx = jax.lax.broadcasted_iota(jnp.int32, (1, N_CONES), 1)
    cone = jnp.min(jnp.where(conemax == rowmax, cidx, N_CONES),
                   axis=1, keepdims=True)
    onehot = (cone == cidx).astype(jnp.float32)
    stats = jnp.concatenate([med_ref[...], mad_ref[...]], axis=0)
    st = jax.lax.dot_general(
        stats, onehot, (((1,), (1,)), ((), ())),
        preferred_element_type=jnp.float32)
    out_ref[...] = (scores_ref[...] - st[0:1, :]) / st[1:2, :]


@jax.jit
def kernel(pred_rotmats, scores, grid_rotmats, medians, mads):
    b = pred_rotmats.shape[0]
    pred_flat = pred_rotmats.reshape(b, 9)
    gridT = grid_rotmats.reshape(N_CONES, N_PSI, 9).transpose(
        2, 1, 0).reshape(9, N_SO3)

    out = pl.pallas_call(
        _fused_kernel,
        grid=(b // BLOCK_B,),
        in_specs=[
            pl.BlockSpec((BLOCK_B, 9), lambda i: (i, 0)),
            pl.BlockSpec((9, N_SO3), lambda i: (0, 0)),
            pl.BlockSpec((1, BLOCK_B), lambda i: (0, i)),
            pl.BlockSpec((1, N_CONES), lambda i: (0, 0)),
            pl.BlockSpec((1, N_CONES), lambda i: (0, 0)),
        ],
        out_specs=pl.BlockSpec((1, BLOCK_B), lambda i: (0, i)),
        out_shape=jax.ShapeDtypeStruct((1, b), jnp.float32),
        compiler_params=pltpu.CompilerParams(
            dimension_semantics=("parallel",)),
    )(pred_flat, gridT, scores.reshape(1, b),
      medians.reshape(1, N_CONES), mads.reshape(1, N_CONES))
    return out.reshape(b)


# speedup vs baseline: 4.4312x; 1.0397x over previous
import jax
import jax.numpy as jnp
from jax.experimental import pallas as pl
from jax.experimental.pallas import tpu as pltpu

N_PSI = 24
N_CONES = 192
N_SO3 = N_CONES * N_PSI
BLOCK_B = 1024


def _fused_kernel(pred_ref, grid_ref, scores_ref, med_ref, mad_ref, out_ref):
    simT = jax.lax.dot_general(
        grid_ref[...], pred_ref[...], (((1,), (1,)), ((), ())),
        preferred_element_type=jnp.float32)
    conemax = jnp.max(simT.reshape(N_CONES, N_PSI, simT.shape[1]), axis=1)
    colmax = jnp.max(conemax, axis=0, keepdims=True)
    ridx = jax.lax.broadcasted_iota(jnp.int32, (N_CONES, 1), 0)
    cone = jnp.min(jnp.where(conemax == colmax, ridx, N_CONES),
                   axis=0, keepdims=True)
    onehotT = (cone == ridx).astype(jnp.float32)
    stats = jnp.concatenate([med_ref[...], mad_ref[...]], axis=0)
    st = jnp.dot(stats, onehotT, preferred_element_type=jnp.float32)
    out_ref[...] = (scores_ref[...] - st[0:1, :]) / st[1:2, :]


@jax.jit
def kernel(pred_rotmats, scores, grid_rotmats, medians, mads):
    b = pred_rotmats.shape[0]
    pred_flat = pred_rotmats.reshape(b, 9)
    grid_flat = grid_rotmats.reshape(N_SO3, 9)

    out = pl.pallas_call(
        _fused_kernel,
        grid=(b // BLOCK_B,),
        in_specs=[
            pl.BlockSpec((BLOCK_B, 9), lambda i: (i, 0)),
            pl.BlockSpec((N_SO3, 9), lambda i: (0, 0)),
            pl.BlockSpec((1, BLOCK_B), lambda i: (0, i)),
            pl.BlockSpec((1, N_CONES), lambda i: (0, 0)),
            pl.BlockSpec((1, N_CONES), lambda i: (0, 0)),
        ],
        out_specs=pl.BlockSpec((1, BLOCK_B), lambda i: (0, i)),
        out_shape=jax.ShapeDtypeStruct((1, b), jnp.float32),
        compiler_params=pltpu.CompilerParams(
            dimension_semantics=("parallel",)),
    )(pred_flat, grid_flat, scores.reshape(1, b),
      medians.reshape(1, N_CONES), mads.reshape(1, N_CONES))
    return out.reshape(b)


# speedup vs baseline: 4.8421x; 1.0927x over previous
import jax
import jax.numpy as jnp
from jax.experimental import pallas as pl
from jax.experimental.pallas import tpu as pltpu

N_PSI = 24
N_CONES = 192
N_SO3 = N_CONES * N_PSI
BLOCK_B = 1024


def _fused_kernel(pred_ref, grid_ref, scores_ref, med_ref, mad_ref, out_ref):
    simT = jax.lax.dot_general(
        grid_ref[...], pred_ref[...], (((1,), (1,)), ((), ())),
        preferred_element_type=jnp.float32)
    conemax = simT[0:N_CONES, :]
    for p in range(1, N_PSI):
        conemax = jnp.maximum(conemax, simT[p * N_CONES:(p + 1) * N_CONES, :])
    colmax = jnp.max(conemax, axis=0, keepdims=True)
    ridx = jax.lax.broadcasted_iota(jnp.int32, (N_CONES, 1), 0)
    cone = jnp.min(jnp.where(conemax == colmax, ridx, N_CONES),
                   axis=0, keepdims=True)
    onehotT = (cone == ridx).astype(jnp.float32)
    stats = jnp.concatenate([med_ref[...], mad_ref[...]], axis=0)
    st = jnp.dot(stats, onehotT, preferred_element_type=jnp.float32)
    out_ref[...] = (scores_ref[...] - st[0:1, :]) / st[1:2, :]


@jax.jit
def kernel(pred_rotmats, scores, grid_rotmats, medians, mads):
    b = pred_rotmats.shape[0]
    pred_flat = pred_rotmats.reshape(b, 9)
    grid_flat = grid_rotmats.reshape(N_CONES, N_PSI, 9).transpose(
        1, 0, 2).reshape(N_SO3, 9)

    out = pl.pallas_call(
        _fused_kernel,
        grid=(b // BLOCK_B,),
        in_specs=[
            pl.BlockSpec((BLOCK_B, 9), lambda i: (i, 0)),
            pl.BlockSpec((N_SO3, 9), lambda i: (0, 0)),
            pl.BlockSpec((1, BLOCK_B), lambda i: (0, i)),
            pl.BlockSpec((1, N_CONES), lambda i: (0, 0)),
            pl.BlockSpec((1, N_CONES), lambda i: (0, 0)),
        ],
        out_specs=pl.BlockSpec((1, BLOCK_B), lambda i: (0, i)),
        out_shape=jax.ShapeDtypeStruct((1, b), jnp.float32),
        compiler_params=pltpu.CompilerParams(
            dimension_semantics=("parallel",)),
    )(pred_flat, grid_flat, scores.reshape(1, b),
      medians.reshape(1, N_CONES), mads.reshape(1, N_CONES))
    return out.reshape(b)
